# baseline (speedup 1.0000x reference)
import jax
import jax.numpy as jnp
from jax import lax
from jax.experimental import pallas as pl
from jax.experimental.pallas import tpu as pltpu
from jax.experimental.pallas import tpu_sc as plsc

N_NODES = 10000
D = 128
T = 4
TAU = 2.0
V_TH = 1.0
DELTA = 0.05
STEP_SIZE = 0.1

NC = 2
NS = 16
NW = NC * NS

N_PAD = 10240
ROWS_PER_SUB = N_PAD // NS
DUMMY_DST = 10200

CHUNK = 128
SG = 8
K_DEG = 8


def _sc_mesh():
    return plsc.VectorSubcoreMesh(core_axis_name="c", subcore_axis_name="s")


def _deg_body(dst_hbm, out_hbm, didx_v, ones_v, zbuf_v, sem_s, acc_sh):
    cid = lax.axis_index("c")
    sid = lax.axis_index("s")
    wid = cid * NS + sid
    n_rows = dst_hbm.shape[0] // NW
    n_groups = n_rows // K_DEG

    for i in range(CHUNK // 16):
        ones_v[pl.ds(i * 16, 16)] = jnp.full((16,), 1.0, jnp.float32)
    for i in range(ROWS_PER_SUB // 16):
        zbuf_v[pl.ds(i * 16, 16)] = jnp.zeros((16,), jnp.float32)

    row0 = sid * ROWS_PER_SUB
    pltpu.sync_copy(zbuf_v, acc_sh.at[pl.ds(row0, ROWS_PER_SUB)])
    pltpu.sync_copy(dst_hbm.at[pl.ds(wid * n_rows, n_rows), :], didx_v)
    plsc.subcore_barrier()

    def group(g, carry):
        base = g * K_DEG
        for b in range(K_DEG):
            pltpu.async_copy(ones_v, acc_sh.at[didx_v.at[base + b]],
                             sem_s, add=True)
        for b in range(K_DEG):
            pltpu.make_async_copy(ones_v, acc_sh.at[didx_v.at[base + b]],
                                  sem_s).wait()
        return carry

    lax.fori_loop(0, n_groups, group, 0)
    plsc.subcore_barrier()

    pltpu.sync_copy(acc_sh.at[pl.ds(row0, ROWS_PER_SUB)],
                    out_hbm.at[pl.ds(cid * N_PAD + row0, ROWS_PER_SUB)])


def _deg_counts(dst2d):
    n_rows_w = dst2d.shape[0] // NW
    kfn = pl.kernel(
        _deg_body,
        out_type=jax.ShapeDtypeStruct((NC * N_PAD,), jnp.float32),
        mesh=_sc_mesh(),
        scratch_types=[
            pltpu.VMEM((n_rows_w, CHUNK), jnp.int32),
            pltpu.VMEM((CHUNK,), jnp.float32),
            pltpu.VMEM((ROWS_PER_SUB,), jnp.float32),
            pltpu.SemaphoreType.DMA,
            pltpu.VMEM_SHARED((N_PAD,), jnp.float32),
        ],
    )
    return kfn(dst2d).reshape(NC, N_PAD)[:, :, None]


AGG_CHUNK = 80


def _agg_body(src_hbm, dst_hbm, xs_hbm, zeros_hbm, out_hbm,
              sidx_v, didx_v, rows_v, sem, sem_i, acc_sh):
    cid = lax.axis_index("c")
    sid = lax.axis_index("s")
    wid = cid * NS + sid
    e_per_w = src_hbm.shape[0] // NW
    n_chunks = e_per_w // AGG_CHUNK

    row0 = sid * ROWS_PER_SUB
    pltpu.sync_copy(zeros_hbm.at[pl.ds(row0, ROWS_PER_SUB), :],
                    acc_sh.at[pl.ds(row0, ROWS_PER_SUB), :])
    plsc.subcore_barrier()

    base = wid * e_per_w

    pltpu.sync_copy(src_hbm.at[pl.ds(base, AGG_CHUNK)], sidx_v.at[0])
    pltpu.sync_copy(dst_hbm.at[pl.ds(base, AGG_CHUNK)], didx_v.at[0])
    pltpu.async_copy(xs_hbm.at[sidx_v.at[0]], rows_v.at[0], sem)
    pltpu.async_copy(src_hbm.at[pl.ds(base + AGG_CHUNK, AGG_CHUNK)],
                     sidx_v.at[1], sem_i)
    pltpu.async_copy(dst_hbm.at[pl.ds(base + AGG_CHUNK, AGG_CHUNK)],
                     didx_v.at[1], sem_i)

    def chunk(c, carry):
        cb = lax.rem(c, 2)
        nb = lax.rem(c + 1, 2)

        @pl.when(c + 1 < n_chunks)
        def _():
            pltpu.make_async_copy(src_hbm.at[pl.ds(base, AGG_CHUNK)],
                                  sidx_v.at[0], sem_i).wait()
            pltpu.make_async_copy(src_hbm.at[pl.ds(base, AGG_CHUNK)],
                                  didx_v.at[0], sem_i).wait()
            pltpu.async_copy(xs_hbm.at[sidx_v.at[nb]], rows_v.at[nb], sem)

        pltpu.make_async_copy(xs_hbm.at[sidx_v.at[0]], rows_v.at[0],
                              sem).wait()
        pltpu.sync_copy(rows_v.at[cb], acc_sh.at[didx_v.at[cb]], add=True)

        @pl.when(c + 2 < n_chunks)
        def _():
            off = base + (c + 2) * AGG_CHUNK
            pltpu.async_copy(src_hbm.at[pl.ds(off, AGG_CHUNK)],
                             sidx_v.at[cb], sem_i)
            pltpu.async_copy(dst_hbm.at[pl.ds(off, AGG_CHUNK)],
                             didx_v.at[cb], sem_i)

        return carry

    lax.fori_loop(0, n_chunks, chunk, 0)
    plsc.subcore_barrier()

    pltpu.sync_copy(acc_sh.at[pl.ds(row0, ROWS_PER_SUB), :],
                    out_hbm.at[cid, pl.ds(row0, ROWS_PER_SUB), :])


def _aggregate(src, dst, xs):
    kfn = pl.kernel(
        _agg_body,
        out_type=jax.ShapeDtypeStruct((NC, N_PAD, D), jnp.float32),
        mesh=_sc_mesh(),
        scratch_types=[
            pltpu.VMEM((2, AGG_CHUNK), jnp.int32),
            pltpu.VMEM((2, AGG_CHUNK), jnp.int32),
            pltpu.VMEM((2, AGG_CHUNK, D), jnp.float32),
            pltpu.SemaphoreType.DMA,
            pltpu.SemaphoreType.DMA,
            pltpu.VMEM_SHARED((N_PAD, D), jnp.float32),
        ],
    )
    zeros = jnp.zeros((N_PAD, D), jnp.float32)
    return kfn(src, dst, xs, zeros)


def _scale_body(x_ref, degp_ref, xs_ref):
    deg = 1.0 + degp_ref[0, :, 0:1] + degp_ref[1, :, 0:1]
    dinv = lax.rsqrt(deg)
    xs_ref[...] = x_ref[...] * dinv


def _scale(x, degp, rows):
    grid = (N_NODES // rows,)
    return pl.pallas_call(
        _scale_body,
        grid=grid,
        in_specs=[
            pl.BlockSpec((rows, D), lambda i: (i, 0)),
            pl.BlockSpec((NC, rows, 1), lambda i: (0, i, 0)),
        ],
        out_specs=pl.BlockSpec((rows, D), lambda i: (i, 0)),
        out_shape=jax.ShapeDtypeStruct((N_NODES, D), jnp.float32),
    )(x, degp)


def _final_body(aggp_ref, x_ref, degp_ref, w_ref, o_ref, z_ref):
    deg = 1.0 + degp_ref[0, :, 0:1] + degp_ref[1, :, 0:1]
    dinv = lax.rsqrt(deg)
    agg = aggp_ref[0] + aggp_ref[1]
    u = dinv * agg + (dinv * dinv) * x_ref[...]
    h = jnp.dot(u, w_ref[...], preferred_element_type=jnp.float32)

    dx = h * STEP_SIZE
    v = jnp.zeros_like(h)
    z = jnp.zeros_like(h)
    for t in range(T):
        v = v + (h - v) * (1.0 / TAU)
        o = (v >= V_TH).astype(jnp.float32)
        v = v - o * (V_TH - DELTA)
        z = z + dx * o
        o_ref[t] = o
        z_ref[t] = z


def _final(aggp, x, degp, W, rows):
    grid = (N_NODES // rows,)
    out_shape = jax.ShapeDtypeStruct((T, N_NODES, D), jnp.float32)
    return pl.pallas_call(
        _final_body,
        grid=grid,
        in_specs=[
            pl.BlockSpec((NC, rows, D), lambda i: (0, i, 0)),
            pl.BlockSpec((rows, D), lambda i: (i, 0)),
            pl.BlockSpec((NC, rows, 1), lambda i: (0, i, 0)),
            pl.BlockSpec((D, D), lambda i: (0, 0)),
        ],
        out_specs=[
            pl.BlockSpec((T, rows, D), lambda i: (0, i, 0)),
            pl.BlockSpec((T, rows, D), lambda i: (0, i, 0)),
        ],
        out_shape=[out_shape, out_shape],
    )(aggp, x, degp, W)


def kernel(x, edge_index, W):
    src = edge_index[0].astype(jnp.int32)
    dst = edge_index[1].astype(jnp.int32)

    e = src.shape[0]
    rows_w = -(-e // (NW * CHUNK * K_DEG)) * K_DEG
    e_pad = NW * rows_w * CHUNK
    pad = e_pad - e
    src2d = jnp.concatenate(
        [src, jnp.zeros((pad,), jnp.int32)]).reshape(-1, CHUNK)
    dummy = N_NODES + (jnp.arange(pad, dtype=jnp.int32) % (N_PAD - N_NODES))
    dst2d = jnp.concatenate([dst, dummy]).reshape(-1, CHUNK)

    degp = _deg_counts(dst2d)
    xs = _scale(x, degp, rows=1000)
    aggp = _aggregate(src2d.reshape(-1), dst2d.reshape(-1), xs)
    o_seq, z_seq = _final(aggp, x, degp, W, rows=1000)
    return (o_seq, z_seq)


# speedup vs baseline: 2.4573x; 2.4573x over previous
import jax
import jax.numpy as jnp
from jax import lax
from jax.experimental import pallas as pl
from jax.experimental.pallas import tpu as pltpu
from jax.experimental.pallas import tpu_sc as plsc

N_NODES = 10000
D = 128
T = 4
TAU = 2.0
V_TH = 1.0
DELTA = 0.05
STEP_SIZE = 0.1

NC = 2
NS = 16
NW = NC * NS

N_PAD = 10240
ROWS_PER_SUB = N_PAD // NS
DUMMY_DST = 10200

CHUNK = 128
SG = 8
K_DEG = 8


def _sc_mesh():
    return plsc.VectorSubcoreMesh(core_axis_name="c", subcore_axis_name="s")


def _deg_body(dst_hbm, out_hbm, didx_v, ones_v, zbuf_v, sem_s, acc_sh):
    cid = lax.axis_index("c")
    sid = lax.axis_index("s")
    wid = cid * NS + sid
    n_rows = dst_hbm.shape[0] // NW
    n_groups = n_rows // K_DEG

    for i in range(CHUNK // 16):
        ones_v[pl.ds(i * 16, 16)] = jnp.full((16,), 1.0, jnp.float32)
    for i in range(ROWS_PER_SUB // 16):
        zbuf_v[pl.ds(i * 16, 16)] = jnp.zeros((16,), jnp.float32)

    row0 = sid * ROWS_PER_SUB
    pltpu.sync_copy(zbuf_v, acc_sh.at[pl.ds(row0, ROWS_PER_SUB)])
    pltpu.sync_copy(dst_hbm.at[pl.ds(wid * n_rows, n_rows), :], didx_v)
    plsc.subcore_barrier()

    def group(g, carry):
        base = g * K_DEG
        for b in range(K_DEG):
            pltpu.async_copy(ones_v, acc_sh.at[didx_v.at[base + b]],
                             sem_s, add=True)
        for b in range(K_DEG):
            pltpu.make_async_copy(ones_v, acc_sh.at[didx_v.at[base + b]],
                                  sem_s).wait()
        return carry

    lax.fori_loop(0, n_groups, group, 0)
    plsc.subcore_barrier()

    pltpu.sync_copy(acc_sh.at[pl.ds(row0, ROWS_PER_SUB)],
                    out_hbm.at[pl.ds(cid * N_PAD + row0, ROWS_PER_SUB)])


def _deg_counts(dst2d):
    n_rows_w = dst2d.shape[0] // NW
    kfn = pl.kernel(
        _deg_body,
        out_type=jax.ShapeDtypeStruct((NC * N_PAD,), jnp.float32),
        mesh=_sc_mesh(),
        scratch_types=[
            pltpu.VMEM((n_rows_w, CHUNK), jnp.int32),
            pltpu.VMEM((CHUNK,), jnp.float32),
            pltpu.VMEM((ROWS_PER_SUB,), jnp.float32),
            pltpu.SemaphoreType.DMA,
            pltpu.VMEM_SHARED((N_PAD,), jnp.float32),
        ],
    )
    return kfn(dst2d).reshape(NC, N_PAD)[:, :, None]


AGG_CHUNK = 80


def _agg_body(src_hbm, dst_hbm, xs_hbm, zeros_hbm, out_hbm,
              sidx_v, didx_v, rows_v, sem, sem_i, acc_sh):
    cid = lax.axis_index("c")
    sid = lax.axis_index("s")
    wid = cid * NS + sid
    e_per_w = src_hbm.shape[0] // NW
    n_chunks = e_per_w // AGG_CHUNK

    row0 = sid * ROWS_PER_SUB
    pltpu.sync_copy(zeros_hbm.at[pl.ds(row0, ROWS_PER_SUB), :],
                    acc_sh.at[pl.ds(row0, ROWS_PER_SUB), :])
    plsc.subcore_barrier()

    base = wid * e_per_w

    pltpu.sync_copy(src_hbm.at[pl.ds(base, AGG_CHUNK)], sidx_v.at[0])
    pltpu.sync_copy(dst_hbm.at[pl.ds(base, AGG_CHUNK)], didx_v.at[0])
    pltpu.async_copy(xs_hbm.at[sidx_v.at[0]], rows_v.at[0], sem)
    pltpu.async_copy(src_hbm.at[pl.ds(base + AGG_CHUNK, AGG_CHUNK)],
                     sidx_v.at[1], sem_i)
    pltpu.async_copy(dst_hbm.at[pl.ds(base + AGG_CHUNK, AGG_CHUNK)],
                     didx_v.at[1], sem_i)

    def chunk(c, carry):
        cb = lax.rem(c, 2)
        nb = lax.rem(c + 1, 2)

        @pl.when(c + 1 < n_chunks)
        def _():
            pltpu.make_async_copy(src_hbm.at[pl.ds(base, AGG_CHUNK)],
                                  sidx_v.at[0], sem_i).wait()
            pltpu.make_async_copy(src_hbm.at[pl.ds(base, AGG_CHUNK)],
                                  didx_v.at[0], sem_i).wait()
            pltpu.async_copy(xs_hbm.at[sidx_v.at[nb]], rows_v.at[nb], sem)

        pltpu.make_async_copy(xs_hbm.at[sidx_v.at[0]], rows_v.at[0],
                              sem).wait()
        pltpu.sync_copy(rows_v.at[cb], acc_sh.at[didx_v.at[cb]], add=True)

        @pl.when(c + 2 < n_chunks)
        def _():
            off = base + (c + 2) * AGG_CHUNK
            pltpu.async_copy(src_hbm.at[pl.ds(off, AGG_CHUNK)],
                             sidx_v.at[cb], sem_i)
            pltpu.async_copy(dst_hbm.at[pl.ds(off, AGG_CHUNK)],
                             didx_v.at[cb], sem_i)

        return carry

    lax.fori_loop(0, n_chunks, chunk, 0)
    plsc.subcore_barrier()

    pltpu.sync_copy(acc_sh.at[pl.ds(row0, ROWS_PER_SUB), :],
                    out_hbm.at[cid, pl.ds(row0, ROWS_PER_SUB), :])


def _aggregate(src, dst, xs):
    kfn = pl.kernel(
        _agg_body,
        out_type=jax.ShapeDtypeStruct((NC, N_PAD, D), jnp.float32),
        mesh=_sc_mesh(),
        scratch_types=[
            pltpu.VMEM((2, AGG_CHUNK), jnp.int32),
            pltpu.VMEM((2, AGG_CHUNK), jnp.int32),
            pltpu.VMEM((2, AGG_CHUNK, D), jnp.float32),
            pltpu.SemaphoreType.DMA,
            pltpu.SemaphoreType.DMA,
            pltpu.VMEM_SHARED((N_PAD, D), jnp.float32),
        ],
    )
    zeros = jnp.zeros((N_PAD, D), jnp.float32)
    return kfn(src, dst, xs, zeros)


def _scale_body(x_ref, degp_ref, xs_ref):
    deg = 1.0 + degp_ref[0, :, 0:1] + degp_ref[1, :, 0:1]
    dinv = lax.rsqrt(deg)
    xs_ref[...] = x_ref[...] * dinv


def _scale(x, degp, rows):
    grid = (N_NODES // rows,)
    return pl.pallas_call(
        _scale_body,
        grid=grid,
        in_specs=[
            pl.BlockSpec((rows, D), lambda i: (i, 0)),
            pl.BlockSpec((NC, rows, 1), lambda i: (0, i, 0)),
        ],
        out_specs=pl.BlockSpec((rows, D), lambda i: (i, 0)),
        out_shape=jax.ShapeDtypeStruct((N_NODES, D), jnp.float32),
    )(x, degp)


def _final_body(aggp_ref, x_ref, degp_ref, w_ref, o_ref, z_ref):
    deg = 1.0 + degp_ref[0, :, 0:1] + degp_ref[1, :, 0:1]
    dinv = lax.rsqrt(deg)
    agg = aggp_ref[0] + aggp_ref[1]
    u = dinv * agg + (dinv * dinv) * x_ref[...]
    h = jnp.dot(u, w_ref[...], preferred_element_type=jnp.float32)

    dx = h * STEP_SIZE
    v = jnp.zeros_like(h)
    z = jnp.zeros_like(h)
    for t in range(T):
        v = v + (h - v) * (1.0 / TAU)
        o = (v >= V_TH).astype(jnp.float32)
        v = v - o * (V_TH - DELTA)
        z = z + dx * o
        o_ref[t] = o
        z_ref[t] = z


def _final(aggp, x, degp, W, rows):
    grid = (N_NODES // rows,)
    out_shape = jax.ShapeDtypeStruct((T, N_NODES, D), jnp.float32)
    return pl.pallas_call(
        _final_body,
        grid=grid,
        in_specs=[
            pl.BlockSpec((NC, rows, D), lambda i: (0, i, 0)),
            pl.BlockSpec((rows, D), lambda i: (i, 0)),
            pl.BlockSpec((NC, rows, 1), lambda i: (0, i, 0)),
            pl.BlockSpec((D, D), lambda i: (0, 0)),
        ],
        out_specs=[
            pl.BlockSpec((T, rows, D), lambda i: (0, i, 0)),
            pl.BlockSpec((T, rows, D), lambda i: (0, i, 0)),
        ],
        out_shape=[out_shape, out_shape],
    )(aggp, x, degp, W)


def kernel(x, edge_index, W):
    src = edge_index[0].astype(jnp.int32)
    dst = edge_index[1].astype(jnp.int32)

    e = src.shape[0]
    rows_w = -(-e // (NW * CHUNK * K_DEG)) * K_DEG
    e_pad = NW * rows_w * CHUNK
    pad = e_pad - e
    src2d = jnp.concatenate(
        [src, jnp.zeros((pad,), jnp.int32)]).reshape(-1, CHUNK)
    dummy = N_NODES + (jnp.arange(pad, dtype=jnp.int32) % (N_PAD - N_NODES))
    dst2d = jnp.concatenate([dst, dummy]).reshape(-1, CHUNK)

    degp = _deg_counts(dst2d)
    xs = _scale(x, degp, rows=1000)
    aggp = _aggregate(src, dst, xs)
    o_seq, z_seq = _final(aggp, x, degp, W, rows=1000)
    return (o_seq, z_seq)


# speedup vs baseline: 2.7203x; 1.1070x over previous
import jax
import jax.numpy as jnp
from jax import lax
from jax.experimental import pallas as pl
from jax.experimental.pallas import tpu as pltpu
from jax.experimental.pallas import tpu_sc as plsc

N_NODES = 10000
D = 128
T = 4
TAU = 2.0
V_TH = 1.0
DELTA = 0.05
STEP_SIZE = 0.1

NC = 2
NS = 16
NW = NC * NS

N_PAD = 10240
ROWS_PER_SUB = N_PAD // NS
DUMMY_DST = 10200

CHUNK = 128
SG = 8
K_DEG = 8


def _sc_mesh():
    return plsc.VectorSubcoreMesh(core_axis_name="c", subcore_axis_name="s")


def _deg_body(dst_hbm, out_hbm, didx_v, ones_v, zbuf_v, sem_s, acc_sh):
    cid = lax.axis_index("c")
    sid = lax.axis_index("s")
    wid = cid * NS + sid
    n_rows = dst_hbm.shape[0] // NW
    n_groups = n_rows // K_DEG

    for i in range(CHUNK // 16):
        ones_v[pl.ds(i * 16, 16)] = jnp.full((16,), 1.0, jnp.float32)
    for i in range(ROWS_PER_SUB // 16):
        zbuf_v[pl.ds(i * 16, 16)] = jnp.zeros((16,), jnp.float32)

    row0 = sid * ROWS_PER_SUB
    pltpu.sync_copy(zbuf_v, acc_sh.at[pl.ds(row0, ROWS_PER_SUB)])
    pltpu.sync_copy(dst_hbm.at[pl.ds(wid * n_rows, n_rows), :], didx_v)
    plsc.subcore_barrier()

    def group(g, carry):
        base = g * K_DEG
        for b in range(K_DEG):
            pltpu.async_copy(ones_v, acc_sh.at[didx_v.at[base + b]],
                             sem_s, add=True)
        for b in range(K_DEG):
            pltpu.make_async_copy(ones_v, acc_sh.at[didx_v.at[base + b]],
                                  sem_s).wait()
        return carry

    lax.fori_loop(0, n_groups, group, 0)
    plsc.subcore_barrier()

    pltpu.sync_copy(acc_sh.at[pl.ds(row0, ROWS_PER_SUB)],
                    out_hbm.at[pl.ds(cid * N_PAD + row0, ROWS_PER_SUB)])


def _deg_counts(dst2d):
    n_rows_w = dst2d.shape[0] // NW
    kfn = pl.kernel(
        _deg_body,
        out_type=jax.ShapeDtypeStruct((NC * N_PAD,), jnp.float32),
        mesh=_sc_mesh(),
        scratch_types=[
            pltpu.VMEM((n_rows_w, CHUNK), jnp.int32),
            pltpu.VMEM((CHUNK,), jnp.float32),
            pltpu.VMEM((ROWS_PER_SUB,), jnp.float32),
            pltpu.SemaphoreType.DMA,
            pltpu.VMEM_SHARED((N_PAD,), jnp.float32),
        ],
    )
    return kfn(dst2d).reshape(NC, N_PAD)[:, :, None]


AGG_CHUNK = 80


def _agg_body(src_hbm, dst_hbm, xs_hbm, zeros_hbm, out_hbm,
              sidx_v, didx_v, rows_v, sem, sem_i, sem_s, acc_sh):
    cid = lax.axis_index("c")
    sid = lax.axis_index("s")
    wid = cid * NS + sid
    e_per_w = src_hbm.shape[0] // NW
    n_chunks = e_per_w // AGG_CHUNK

    row0 = sid * ROWS_PER_SUB
    pltpu.sync_copy(zeros_hbm.at[pl.ds(row0, ROWS_PER_SUB), :],
                    acc_sh.at[pl.ds(row0, ROWS_PER_SUB), :])
    plsc.subcore_barrier()

    base = wid * e_per_w

    pltpu.sync_copy(src_hbm.at[pl.ds(base, AGG_CHUNK)], sidx_v.at[0])
    pltpu.sync_copy(dst_hbm.at[pl.ds(base, AGG_CHUNK)], didx_v.at[0])
    pltpu.async_copy(xs_hbm.at[sidx_v.at[0]], rows_v.at[0], sem)
    pltpu.async_copy(src_hbm.at[pl.ds(base + AGG_CHUNK, AGG_CHUNK)],
                     sidx_v.at[1], sem_i)
    pltpu.async_copy(dst_hbm.at[pl.ds(base + AGG_CHUNK, AGG_CHUNK)],
                     didx_v.at[1], sem_i)

    def _drain_s():
        pltpu.make_async_copy(rows_v.at[0], acc_sh.at[didx_v.at[0]],
                              sem_s).wait()

    def chunk(c, carry):
        @pl.when(c >= 2)
        def _():
            _drain_s()

        @pl.when(c + 1 < n_chunks)
        def _():
            pltpu.make_async_copy(src_hbm.at[pl.ds(base, AGG_CHUNK)],
                                  sidx_v.at[0], sem_i).wait()
            pltpu.make_async_copy(src_hbm.at[pl.ds(base, AGG_CHUNK)],
                                  didx_v.at[0], sem_i).wait()
            pltpu.async_copy(xs_hbm.at[sidx_v.at[lax.rem(c + 1, 4)]],
                             rows_v.at[lax.rem(c + 1, 3)], sem)

        pltpu.make_async_copy(xs_hbm.at[sidx_v.at[0]], rows_v.at[0],
                              sem).wait()
        pltpu.async_copy(rows_v.at[lax.rem(c, 3)],
                         acc_sh.at[didx_v.at[lax.rem(c, 4)]],
                         sem_s, add=True)

        @pl.when(c + 2 < n_chunks)
        def _():
            off = base + (c + 2) * AGG_CHUNK
            slot = lax.rem(c + 2, 4)
            pltpu.async_copy(src_hbm.at[pl.ds(off, AGG_CHUNK)],
                             sidx_v.at[slot], sem_i)
            pltpu.async_copy(dst_hbm.at[pl.ds(off, AGG_CHUNK)],
                             didx_v.at[slot], sem_i)

        return carry

    lax.fori_loop(0, n_chunks, chunk, 0)
    _drain_s()
    _drain_s()
    plsc.subcore_barrier()

    pltpu.sync_copy(acc_sh.at[pl.ds(row0, ROWS_PER_SUB), :],
                    out_hbm.at[cid, pl.ds(row0, ROWS_PER_SUB), :])


def _aggregate(src, dst, xs):
    kfn = pl.kernel(
        _agg_body,
        out_type=jax.ShapeDtypeStruct((NC, N_PAD, D), jnp.float32),
        mesh=_sc_mesh(),
        scratch_types=[
            pltpu.VMEM((4, AGG_CHUNK), jnp.int32),
            pltpu.VMEM((4, AGG_CHUNK), jnp.int32),
            pltpu.VMEM((3, AGG_CHUNK, D), jnp.float32),
            pltpu.SemaphoreType.DMA,
            pltpu.SemaphoreType.DMA,
            pltpu.SemaphoreType.DMA,
            pltpu.VMEM_SHARED((N_PAD, D), jnp.float32),
        ],
    )
    zeros = jnp.zeros((N_PAD, D), jnp.float32)
    return kfn(src, dst, xs, zeros)


def _scale_body(x_ref, degp_ref, xs_ref):
    deg = 1.0 + degp_ref[0, :, 0:1] + degp_ref[1, :, 0:1]
    dinv = lax.rsqrt(deg)
    xs_ref[...] = x_ref[...] * dinv


def _scale(x, degp, rows):
    grid = (N_NODES // rows,)
    return pl.pallas_call(
        _scale_body,
        grid=grid,
        in_specs=[
            pl.BlockSpec((rows, D), lambda i: (i, 0)),
            pl.BlockSpec((NC, rows, 1), lambda i: (0, i, 0)),
        ],
        out_specs=pl.BlockSpec((rows, D), lambda i: (i, 0)),
        out_shape=jax.ShapeDtypeStruct((N_NODES, D), jnp.float32),
    )(x, degp)


def _final_body(aggp_ref, x_ref, degp_ref, w_ref, o_ref, z_ref):
    deg = 1.0 + degp_ref[0, :, 0:1] + degp_ref[1, :, 0:1]
    dinv = lax.rsqrt(deg)
    agg = aggp_ref[0] + aggp_ref[1]
    u = dinv * agg + (dinv * dinv) * x_ref[...]
    h = jnp.dot(u, w_ref[...], preferred_element_type=jnp.float32)

    dx = h * STEP_SIZE
    v = jnp.zeros_like(h)
    z = jnp.zeros_like(h)
    for t in range(T):
        v = v + (h - v) * (1.0 / TAU)
        o = (v >= V_TH).astype(jnp.float32)
        v = v - o * (V_TH - DELTA)
        z = z + dx * o
        o_ref[t] = o
        z_ref[t] = z


def _final(aggp, x, degp, W, rows):
    grid = (N_NODES // rows,)
    out_shape = jax.ShapeDtypeStruct((T, N_NODES, D), jnp.float32)
    return pl.pallas_call(
        _final_body,
        grid=grid,
        in_specs=[
            pl.BlockSpec((NC, rows, D), lambda i: (0, i, 0)),
            pl.BlockSpec((rows, D), lambda i: (i, 0)),
            pl.BlockSpec((NC, rows, 1), lambda i: (0, i, 0)),
            pl.BlockSpec((D, D), lambda i: (0, 0)),
        ],
        out_specs=[
            pl.BlockSpec((T, rows, D), lambda i: (0, i, 0)),
            pl.BlockSpec((T, rows, D), lambda i: (0, i, 0)),
        ],
        out_shape=[out_shape, out_shape],
    )(aggp, x, degp, W)


def kernel(x, edge_index, W):
    src = edge_index[0].astype(jnp.int32)
    dst = edge_index[1].astype(jnp.int32)

    e = src.shape[0]
    rows_w = -(-e // (NW * CHUNK * K_DEG)) * K_DEG
    e_pad = NW * rows_w * CHUNK
    pad = e_pad - e
    src2d = jnp.concatenate(
        [src, jnp.zeros((pad,), jnp.int32)]).reshape(-1, CHUNK)
    dummy = N_NODES + (jnp.arange(pad, dtype=jnp.int32) % (N_PAD - N_NODES))
    dst2d = jnp.concatenate([dst, dummy]).reshape(-1, CHUNK)

    degp = _deg_counts(dst2d)
    xs = _scale(x, degp, rows=1000)
    aggp = _aggregate(src, dst, xs)
    o_seq, z_seq = _final(aggp, x, degp, W, rows=1000)
    return (o_seq, z_seq)
